# baseline (speedup 1.0000x reference)
import functools

import jax
import jax.numpy as jnp
from jax import lax
from jax.experimental import pallas as pl
from jax.experimental.pallas import tpu as pltpu
from jax.experimental.pallas import tpu_sc as plsc

NUM_CORES = 2
NUM_SUBCORES = 16
LANES = 16
NUM_WORKERS = NUM_CORES * NUM_SUBCORES

BATCH = 16384
EMBED = 32
NROWS = 1000000
BPW = BATCH // NUM_WORKERS
SLICES = BPW // LANES


def _sc_body(uidx_h, midx_h, uemb_h, memb_h, ubias_h, mbias_h,
             partials_h, bsum_h,
             uidx_v, midx_v, uofs_v, mofs_v, ug_v, mg_v,
             ubias_v, mbias_v, bsum_v, acc_v,
             sem_u, sem_m, sem_ub, sem_mb):
    wid = lax.axis_index("s") * NUM_CORES + lax.axis_index("c")
    base = wid * BPW

    pltpu.sync_copy(uidx_h.at[pl.ds(base, BPW)], uidx_v)
    pltpu.sync_copy(midx_h.at[pl.ds(base, BPW)], midx_v)

    cp_ub = pltpu.async_copy(ubias_h.at[uidx_v], ubias_v, sem_ub)
    cp_mb = pltpu.async_copy(mbias_h.at[midx_v], mbias_v, sem_mb)

    def build(j, _):
        sl = pl.ds(j * LANES, LANES)
        ubase = uidx_v[sl]
        mbase = midx_v[sl]
        for e in range(EMBED):
            uofs_v[pl.ds(e * BPW + j * LANES, LANES)] = ubase + e * NROWS
            mofs_v[pl.ds(e * BPW + j * LANES, LANES)] = mbase + e * NROWS
        return 0

    lax.fori_loop(0, SLICES, build, 0, unroll=False)

    cp_u = pltpu.async_copy(uemb_h.at[uofs_v], ug_v, sem_u)
    cp_m = pltpu.async_copy(memb_h.at[mofs_v], mg_v, sem_m)

    cp_ub.wait()
    cp_mb.wait()

    def bias_body(j, _):
        sl = pl.ds(j * LANES, LANES)
        bsum_v[sl] = ubias_v[sl] + mbias_v[sl]
        return 0

    lax.fori_loop(0, SLICES, bias_body, 0)
    pltpu.sync_copy(bsum_v, bsum_h.at[pl.ds(base, BPW)])

    cp_u.wait()
    cp_m.wait()

    def dot_body(i, acc):
        sl = pl.ds(i * LANES, LANES)
        return acc + ug_v[sl] * mg_v[sl]

    acc = lax.fori_loop(0, BATCH // LANES, dot_body,
                        jnp.zeros((LANES,), jnp.float32))
    acc_v[...] = acc
    pltpu.sync_copy(acc_v, partials_h.at[pl.ds(wid * LANES, LANES)])


@functools.partial(
    pl.kernel,
    out_type=(
        jax.ShapeDtypeStruct((NUM_WORKERS * LANES,), jnp.float32),
        jax.ShapeDtypeStruct((BATCH,), jnp.float32),
    ),
    mesh=plsc.VectorSubcoreMesh(core_axis_name="c", subcore_axis_name="s"),
    compiler_params=pltpu.CompilerParams(use_tc_tiling_on_sc=False),
    scratch_types=(
        pltpu.VMEM((BPW,), jnp.int32),
        pltpu.VMEM((BPW,), jnp.int32),
        pltpu.VMEM((BATCH,), jnp.int32),
        pltpu.VMEM((BATCH,), jnp.int32),
        pltpu.VMEM((BATCH,), jnp.float32),
        pltpu.VMEM((BATCH,), jnp.float32),
        pltpu.VMEM((BPW,), jnp.float32),
        pltpu.VMEM((BPW,), jnp.float32),
        pltpu.VMEM((BPW,), jnp.float32),
        pltpu.VMEM((LANES,), jnp.float32),
        pltpu.SemaphoreType.DMA,
        pltpu.SemaphoreType.DMA,
        pltpu.SemaphoreType.DMA,
        pltpu.SemaphoreType.DMA,
    ),
)
def _sc_gather(uidx_h, midx_h, uemb_h, memb_h, ubias_h, mbias_h,
               partials_h, bsum_h, *scratch):
    _sc_body(uidx_h, midx_h, uemb_h, memb_h, ubias_h, mbias_h,
             partials_h, bsum_h, *scratch)


def _tc_combine_body(part_ref, bsum_ref, out_ref):
    total = jnp.sum(part_ref[...])
    out_ref[...] = jax.nn.sigmoid(bsum_ref[...] + total)


_tc_combine = pl.pallas_call(
    _tc_combine_body,
    out_shape=jax.ShapeDtypeStruct((BATCH // 128, 128), jnp.float32),
)


def kernel(inputs, user_embedding, user_bias, movie_embedding, movie_bias):
    u_idx = inputs[:, 0]
    m_idx = inputs[:, 1]
    partials, bsum = _sc_gather(
        u_idx, m_idx,
        user_embedding.T.reshape(-1), movie_embedding.T.reshape(-1),
        user_bias.reshape(-1), movie_bias.reshape(-1))
    out = _tc_combine(partials.reshape(4, 128), bsum.reshape(BATCH // 128, 128))
    return out.reshape(BATCH, 1)


# speedup vs baseline: 5.8237x; 5.8237x over previous
import functools

import jax
import jax.numpy as jnp
from jax import lax
from jax.experimental import pallas as pl
from jax.experimental.pallas import tpu as pltpu
from jax.experimental.pallas import tpu_sc as plsc

NUM_CORES = 2
NUM_SUBCORES = 16
LANES = 16
NUM_WORKERS = NUM_CORES * NUM_SUBCORES

BATCH = 16384
EMBED = 32
BPW = BATCH // NUM_WORKERS
SLICES = BPW // LANES

_MESH = plsc.VectorSubcoreMesh(core_axis_name="c", subcore_axis_name="s")
_NOTC = pltpu.CompilerParams(use_tc_tiling_on_sc=False)


def _sc_user_body(uidx_h, midx_h, uemb_h, ubias_h, mbias_h,
                  urows_out_h, bsum_h,
                  uidx_v, midx_v, urows_v, ubias_v, mbias_v, bsum_v,
                  sem_u, sem_ub, sem_mb):
    wid = lax.axis_index("s") * NUM_CORES + lax.axis_index("c")
    base = wid * BPW

    pltpu.sync_copy(uidx_h.at[pl.ds(base, BPW)], uidx_v)
    pltpu.sync_copy(midx_h.at[pl.ds(base, BPW)], midx_v)

    cp_u = pltpu.async_copy(uemb_h.at[uidx_v], urows_v, sem_u)
    cp_ub = pltpu.async_copy(ubias_h.at[uidx_v], ubias_v, sem_ub)
    cp_mb = pltpu.async_copy(mbias_h.at[midx_v], mbias_v, sem_mb)

    cp_ub.wait()
    cp_mb.wait()

    def bias_body(j, _):
        sl = pl.ds(j * LANES, LANES)
        bsum_v[sl] = ubias_v[sl] + mbias_v[sl]
        return 0

    lax.fori_loop(0, SLICES, bias_body, 0)
    pltpu.sync_copy(bsum_v, bsum_h.at[pl.ds(base, BPW)])

    cp_u.wait()
    pltpu.sync_copy(urows_v, urows_out_h.at[pl.ds(base, BPW)])


@functools.partial(
    pl.kernel,
    out_type=(
        jax.ShapeDtypeStruct((BATCH, EMBED), jnp.float32),
        jax.ShapeDtypeStruct((BATCH,), jnp.float32),
    ),
    mesh=_MESH,
    compiler_params=_NOTC,
    scratch_types=(
        pltpu.VMEM((BPW,), jnp.int32),
        pltpu.VMEM((BPW,), jnp.int32),
        pltpu.VMEM((BPW, EMBED), jnp.float32),
        pltpu.VMEM((BPW,), jnp.float32),
        pltpu.VMEM((BPW,), jnp.float32),
        pltpu.VMEM((BPW,), jnp.float32),
        pltpu.SemaphoreType.DMA,
        pltpu.SemaphoreType.DMA,
        pltpu.SemaphoreType.DMA,
    ),
)
def _sc_user(uidx_h, midx_h, uemb_h, ubias_h, mbias_h,
             urows_out_h, bsum_h, *scratch):
    _sc_user_body(uidx_h, midx_h, uemb_h, ubias_h, mbias_h,
                  urows_out_h, bsum_h, *scratch)


def _sc_movie_body(midx_h, memb_h, urows_h, partials_h,
                   midx_v, mrows_v, urows_v, acc_v, sem_m, sem_ur):
    wid = lax.axis_index("s") * NUM_CORES + lax.axis_index("c")
    base = wid * BPW

    pltpu.sync_copy(midx_h.at[pl.ds(base, BPW)], midx_v)
    cp_m = pltpu.async_copy(memb_h.at[midx_v], mrows_v, sem_m)
    cp_u = pltpu.async_copy(urows_h.at[pl.ds(base, BPW)], urows_v, sem_ur)
    cp_u.wait()
    cp_m.wait()

    def dot_body(i, acc):
        u0 = urows_v[i, pl.ds(0, LANES)]
        m0 = mrows_v[i, pl.ds(0, LANES)]
        u1 = urows_v[i, pl.ds(LANES, LANES)]
        m1 = mrows_v[i, pl.ds(LANES, LANES)]
        return acc + u0 * m0 + u1 * m1

    acc = lax.fori_loop(0, BPW, dot_body, jnp.zeros((LANES,), jnp.float32))
    acc_v[...] = acc
    pltpu.sync_copy(acc_v, partials_h.at[pl.ds(wid * LANES, LANES)])


@functools.partial(
    pl.kernel,
    out_type=jax.ShapeDtypeStruct((NUM_WORKERS * LANES,), jnp.float32),
    mesh=_MESH,
    compiler_params=_NOTC,
    scratch_types=(
        pltpu.VMEM((BPW,), jnp.int32),
        pltpu.VMEM((BPW, EMBED), jnp.float32),
        pltpu.VMEM((BPW, EMBED), jnp.float32),
        pltpu.VMEM((LANES,), jnp.float32),
        pltpu.SemaphoreType.DMA,
        pltpu.SemaphoreType.DMA,
    ),
)
def _sc_movie(midx_h, memb_h, urows_h, partials_h, *scratch):
    _sc_movie_body(midx_h, memb_h, urows_h, partials_h, *scratch)


def _tc_combine_body(part_ref, bsum_ref, out_ref):
    total = jnp.sum(part_ref[...])
    out_ref[...] = jax.nn.sigmoid(bsum_ref[...] + total)


_tc_combine = pl.pallas_call(
    _tc_combine_body,
    out_shape=jax.ShapeDtypeStruct((BATCH // 128, 128), jnp.float32),
)


def kernel(inputs, user_embedding, user_bias, movie_embedding, movie_bias):
    u_idx = inputs[:, 0]
    m_idx = inputs[:, 1]
    urows, bsum = _sc_user(
        u_idx, m_idx, user_embedding,
        user_bias.reshape(-1), movie_bias.reshape(-1))
    partials = _sc_movie(m_idx, movie_embedding, urows)
    out = _tc_combine(partials.reshape(4, 128), bsum.reshape(BATCH // 128, 128))
    return out.reshape(BATCH, 1)
